# baseline (speedup 1.0000x reference)
import functools

import jax
import jax.numpy as jnp
from jax import lax
from jax.experimental import pallas as pl
from jax.experimental.pallas import tpu as pltpu
from jax.experimental.pallas import tpu_sc as plsc

B = 16384
EMB = 32
NC = 2
NS = 16
NW = NC * NS
BW = B // NW
L = 16
CH = 128
NCH = BW // CH


def _mf_body(uid_hbm, iid_hbm, ue_hbm, ie_hbm, ub_hbm, ib_hbm, gb_hbm,
             out_hbm,
             u128_v, i128_v, ue_v, ie_v, ub_v, ib_v, gb_v, out_v, sem):
    wid = lax.axis_index("s") * NC + lax.axis_index("c")
    base = wid * BW

    for c in range(NCH):
        pltpu.sync_copy(uid_hbm.at[pl.ds(base + c * CH, CH)], u128_v.at[c])
        pltpu.sync_copy(iid_hbm.at[pl.ds(base + c * CH, CH)], i128_v.at[c])
    pltpu.sync_copy(gb_hbm, gb_v)

    for c in range(NCH):
        rows = pl.ds(c * CH, CH)
        pltpu.async_copy(ue_hbm.at[u128_v.at[c]], ue_v.at[rows], sem)
        pltpu.async_copy(ie_hbm.at[i128_v.at[c]], ie_v.at[rows], sem)

    def bias_issue(c, _):
        for t in range(CH // L):
            uvec = u128_v[c, pl.ds(t * L, L)]
            ivec = i128_v[c, pl.ds(t * L, L)]
            for e in range(L):
                k = c * CH + t * L + e
                pltpu.async_copy(ub_hbm.at[uvec[e]],
                                 ub_v.at[pl.ds(k * 8, 1)], sem)
                pltpu.async_copy(ib_hbm.at[ivec[e]],
                                 ib_v.at[pl.ds(k * 8, 1)], sem)
        return 0

    lax.fori_loop(0, NCH, bias_issue, 0)

    pltpu.make_async_copy(out_hbm.at[pl.ds(0, BW)], ub_v.at[pl.ds(0, BW)], sem).wait()
    pltpu.make_async_copy(out_hbm.at[pl.ds(0, BW)], ib_v.at[pl.ds(0, BW)], sem).wait()
    for c in range(NCH):
        rows = pl.ds(c * CH, CH)
        pltpu.make_async_copy(ue_hbm.at[pl.ds(0, CH)], ue_v.at[rows], sem).wait()
        pltpu.make_async_copy(ie_hbm.at[pl.ds(0, CH)], ie_v.at[rows], sem).wait()

    gb = gb_v[pl.ds(0, L)]
    iota = lax.iota(jnp.int32, L)

    def group(g, _):
        o = pl.ds(g * L, L)
        rows = g * L + iota
        acc = (gb + plsc.load_gather(ub_v, [rows * 8])
               + plsc.load_gather(ib_v, [rows * 8]))
        for j in range(EMB):
            cols = jnp.full((L,), j, jnp.int32)
            u = plsc.load_gather(ue_v, [rows, cols])
            i = plsc.load_gather(ie_v, [rows, cols])
            acc = acc + u * i
        out_v[o] = acc
        return 0

    lax.fori_loop(0, BW // L, group, 0)
    pltpu.sync_copy(out_v, out_hbm.at[pl.ds(base, BW)])


@functools.partial(jax.jit, static_argnames=())
def kernel(user_id, item_id, user_embedding, item_embedding, user_bias,
           item_bias, global_bias):
    uid = user_id.astype(jnp.int32)
    iid = item_id.astype(jnp.int32)
    gb128 = jnp.broadcast_to(global_bias, (128,)).astype(jnp.float32)

    run = pl.kernel(
        _mf_body,
        out_type=jax.ShapeDtypeStruct((B,), jnp.float32),
        mesh=plsc.VectorSubcoreMesh(
            core_axis_name="c", subcore_axis_name="s",
            num_cores=NC, num_subcores=NS),
        scratch_types=[
            pltpu.VMEM((NCH, CH), jnp.int32),
            pltpu.VMEM((NCH, CH), jnp.int32),
            pltpu.VMEM((BW, EMB), jnp.float32),
            pltpu.VMEM((BW, EMB), jnp.float32),
            pltpu.VMEM((BW * 8,), jnp.float32),
            pltpu.VMEM((BW * 8,), jnp.float32),
            pltpu.VMEM((128,), jnp.float32),
            pltpu.VMEM((BW,), jnp.float32),
            pltpu.SemaphoreType.DMA,
        ],
        compiler_params=pltpu.CompilerParams(
            needs_layout_passes=False, use_tc_tiling_on_sc=False),
    )
    return run(uid, iid, user_embedding, item_embedding, user_bias,
               item_bias, gb128)


# speedup vs baseline: 2.8266x; 2.8266x over previous
import functools

import jax
import jax.numpy as jnp
from jax import lax
from jax.experimental import pallas as pl
from jax.experimental.pallas import tpu as pltpu
from jax.experimental.pallas import tpu_sc as plsc

B = 16384
EMB = 32
NC = 2
NS = 16
NW = NC * NS
BW = B // NW
L = 16
CH = 128
NCH = BW // CH


def _mf_body(uid_hbm, iid_hbm, ue_hbm, ie_hbm, ub_hbm, ib_hbm, gb_hbm,
             out_hbm,
             u128_v, i128_v, ue_v, ie_v, ub_v, ib_v, gb_v, out_v, sem):
    wid = lax.axis_index("s") * NC + lax.axis_index("c")
    base = wid * BW

    for c in range(NCH):
        pltpu.sync_copy(uid_hbm.at[pl.ds(base + c * CH, CH)], u128_v.at[c])
        pltpu.sync_copy(iid_hbm.at[pl.ds(base + c * CH, CH)], i128_v.at[c])
    pltpu.sync_copy(gb_hbm, gb_v)

    for c in range(NCH):
        rows = pl.ds(c * CH, CH)
        pltpu.async_copy(ue_hbm.at[u128_v.at[c]], ue_v.at[rows], sem)
        pltpu.async_copy(ie_hbm.at[i128_v.at[c]], ie_v.at[rows], sem)
        pltpu.async_copy(ub_hbm.at[u128_v.at[c]], ub_v.at[rows], sem)
        pltpu.async_copy(ib_hbm.at[i128_v.at[c]], ib_v.at[rows], sem)

    pltpu.make_async_copy(out_hbm.at[pl.ds(0, BW)], ub_v, sem).wait()
    pltpu.make_async_copy(out_hbm.at[pl.ds(0, BW)], ib_v, sem).wait()
    for c in range(NCH):
        rows = pl.ds(c * CH, CH)
        pltpu.make_async_copy(ue_hbm.at[pl.ds(0, CH)], ue_v.at[rows], sem).wait()
        pltpu.make_async_copy(ie_hbm.at[pl.ds(0, CH)], ie_v.at[rows], sem).wait()

    gb = gb_v[pl.ds(0, L)]
    iota = lax.iota(jnp.int32, L)

    def group(g, _):
        o = pl.ds(g * L, L)
        rows = g * L + iota
        acc = gb + ub_v[o] + ib_v[o]
        for j in range(EMB):
            cols = jnp.full((L,), j, jnp.int32)
            u = plsc.load_gather(ue_v, [rows, cols])
            i = plsc.load_gather(ie_v, [rows, cols])
            acc = acc + u * i
        out_v[o] = acc
        return 0

    lax.fori_loop(0, BW // L, group, 0)
    pltpu.sync_copy(out_v, out_hbm.at[pl.ds(base, BW)])


@functools.partial(jax.jit, static_argnames=())
def kernel(user_id, item_id, user_embedding, item_embedding, user_bias,
           item_bias, global_bias):
    uid = user_id.astype(jnp.int32)
    iid = item_id.astype(jnp.int32)
    ub_lin = user_bias.reshape(-1)
    ib_lin = item_bias.reshape(-1)
    gb128 = jnp.broadcast_to(global_bias, (128,)).astype(jnp.float32)

    run = pl.kernel(
        _mf_body,
        out_type=jax.ShapeDtypeStruct((B,), jnp.float32),
        mesh=plsc.VectorSubcoreMesh(
            core_axis_name="c", subcore_axis_name="s",
            num_cores=NC, num_subcores=NS),
        scratch_types=[
            pltpu.VMEM((NCH, CH), jnp.int32),
            pltpu.VMEM((NCH, CH), jnp.int32),
            pltpu.VMEM((BW, EMB), jnp.float32),
            pltpu.VMEM((BW, EMB), jnp.float32),
            pltpu.VMEM((BW,), jnp.float32),
            pltpu.VMEM((BW,), jnp.float32),
            pltpu.VMEM((128,), jnp.float32),
            pltpu.VMEM((BW,), jnp.float32),
            pltpu.SemaphoreType.DMA,
        ],
        compiler_params=pltpu.CompilerParams(
            needs_layout_passes=False, use_tc_tiling_on_sc=False),
    )
    return run(uid, iid, user_embedding, item_embedding, ub_lin, ib_lin,
               gb128)
